# baseline (speedup 1.0000x reference)
import functools

import jax
import jax.numpy as jnp
from jax import lax
from jax.experimental import pallas as pl
from jax.experimental.pallas import tpu as pltpu
from jax.experimental.pallas import tpu_sc as plsc

NUM_CORES = 2
NUM_SUBCORES = 16
NUM_WORKERS = NUM_CORES * NUM_SUBCORES
LANES = 16

TOPK = 5
NEG = float("-inf")


def _sc_body(acts, out, buf, simbuf, *, bpw, p_dim, sub, blk, pad, unroll):
    nblk = p_dim // blk
    wid = lax.axis_index("s") * NUM_CORES + lax.axis_index("c")
    b0 = wid * bpw
    lane = lax.iota(jnp.int32, LANES)
    ones = jnp.ones((LANES,), jnp.float32)

    def block_body(t, carry):
        bb = t // nblk
        bi = t % nblk
        pltpu.sync_copy(acts.at[b0 + bb, pl.ds(bi * blk, blk)], buf)
        for r in range(blk):
            neg = jnp.full((LANES,), NEG, jnp.float32)
            init = (neg, neg, neg, neg, neg)

            def vec_body(i, v):
                v1, v2, v3, v4, v5 = v
                for j in range(unroll):
                    x = buf[r, i, pl.ds(j * LANES, LANES)]
                    t1 = jnp.maximum(v1, x); x = jnp.minimum(v1, x); v1 = t1
                    t1 = jnp.maximum(v2, x); x = jnp.minimum(v2, x); v2 = t1
                    t1 = jnp.maximum(v3, x); x = jnp.minimum(v3, x); v3 = t1
                    t1 = jnp.maximum(v4, x); x = jnp.minimum(v4, x); v4 = t1
                    v5 = jnp.maximum(v5, x)
                return (v1, v2, v3, v4, v5)

            v1, v2, v3, v4, v5 = lax.fori_loop(0, sub, vec_body, init)

            s = jnp.float32(0.0)
            for _ in range(TOPK):
                m = jnp.max(v1)
                s = s + m
                f = plsc.all_reduce_ffs(v1 == m)
                msk = lane == f
                v1 = jnp.where(msk, v2, v1)
                v2 = jnp.where(msk, v3, v2)
                v3 = jnp.where(msk, v4, v3)
                v4 = jnp.where(msk, v5, v4)
            sim = s * jnp.float32(1.0 / TOPK)

            idx = jnp.full((LANES,), bb * p_dim + bi * blk + r, jnp.int32)
            plsc.store_scatter(simbuf, [idx], ones * sim, mask=lane == 0)
        return carry

    lax.fori_loop(0, bpw * nblk, block_body, 0)
    pltpu.sync_copy(simbuf, out.at[wid])


def _build_sc(bdim, p_dim, sub, lanes, blk, pad, interpret=False):
    bpw = bdim // NUM_WORKERS
    unroll = lanes // LANES
    mesh = plsc.VectorSubcoreMesh(
        core_axis_name="c", subcore_axis_name="s",
        num_cores=NUM_CORES, num_subcores=NUM_SUBCORES)
    return pl.kernel(
        functools.partial(_sc_body, bpw=bpw, p_dim=p_dim, sub=sub, blk=blk,
                          pad=pad, unroll=unroll),
        out_type=jax.ShapeDtypeStruct((NUM_WORKERS, pad), jnp.float32),
        mesh=mesh,
        scratch_types=[
            pltpu.VMEM((blk, sub, lanes), jnp.float32),
            pltpu.VMEM((pad,), jnp.float32),
        ],
        compiler_params=pltpu.CompilerParams(
            needs_layout_passes=False, use_tc_tiling_on_sc=True),
        interpret=interpret,
    )


def _tc_linear(sim_ref, w_ref, b_ref, o_ref):
    s = sim_ref[...].astype(jnp.bfloat16).astype(jnp.float32)
    w = w_ref[...].astype(jnp.bfloat16).astype(jnp.float32)
    o_ref[...] = jnp.sum(s * w, axis=1, keepdims=True) + b_ref[...]


def kernel(prototype_activations, upsampled_activation, W, b):
    B, P = prototype_activations.shape[0], prototype_activations.shape[1]
    sub, lanes = prototype_activations.shape[2], prototype_activations.shape[3]
    rpw = (B // NUM_WORKERS) * P
    pad = (rpw + LANES - 1) // LANES * LANES

    sc = _build_sc(B, P, sub, lanes, blk=10, pad=pad)
    simp = sc(prototype_activations)
    sim = simp[:, :rpw].reshape(B, P)

    logits = pl.pallas_call(
        _tc_linear,
        out_shape=jax.ShapeDtypeStruct((B, 1), jnp.float32),
    )(sim, W, b.reshape(1, 1))
    return logits


# speedup vs baseline: 1.3641x; 1.3641x over previous
import functools

import jax
import jax.numpy as jnp
from jax import lax
from jax.experimental import pallas as pl
from jax.experimental.pallas import tpu as pltpu
from jax.experimental.pallas import tpu_sc as plsc

NUM_CORES = 2
NUM_SUBCORES = 16
NUM_WORKERS = NUM_CORES * NUM_SUBCORES
LANES = 16

TOPK = 5
NEG = float("-inf")


def _sc_body(acts, out, buf, simbuf, sem0, sem1, *, np_, hw, chunk, lanes):
    ng = lanes // LANES
    ch_per_p = hw // chunk
    wid = lax.axis_index("s") * NUM_CORES + lax.axis_index("c")
    base, rem = np_ // NUM_WORKERS, np_ % NUM_WORKERS
    cnt = jnp.where(wid < rem, base + 1, base)
    pstart = wid * base + jnp.minimum(wid, rem)
    sems = (sem0, sem1)
    neg = jnp.full((LANES,), NEG, jnp.float32)

    def chunk_copy(ci, bslot):
        return pltpu.make_async_copy(
            acts.at[pl.ds(ci * chunk, chunk)], buf.at[bslot], sems[bslot])

    first = pstart * ch_per_p
    last = (pstart + cnt) * ch_per_p
    chunk_copy(first, 0).start()

    def p_body(pi, carry):
        acc = tuple(neg for _ in range(TOPK * ng))

        def c2_body(c2, acc):
            for bslot in range(2):
                ci = (pstart + pi) * ch_per_p + c2 * 2 + bslot
                chunk_copy(ci, bslot).wait()

                @pl.when(ci + 1 < last)
                def _():
                    chunk_copy(ci + 1, 1 - bslot).start()

                def pos_body(i, acc):
                    acc = list(acc)
                    for g in range(ng):
                        x = buf[bslot, i, pl.ds(g * LANES, LANES)]
                        v1, v2, v3, v4, v5 = acc[TOPK * g:TOPK * (g + 1)]
                        t1 = jnp.maximum(v1, x); x = jnp.minimum(v1, x); v1 = t1
                        t1 = jnp.maximum(v2, x); x = jnp.minimum(v2, x); v2 = t1
                        t1 = jnp.maximum(v3, x); x = jnp.minimum(v3, x); v3 = t1
                        t1 = jnp.maximum(v4, x); x = jnp.minimum(v4, x); v4 = t1
                        v5 = jnp.maximum(v5, x)
                        acc[TOPK * g:TOPK * (g + 1)] = [v1, v2, v3, v4, v5]
                    return tuple(acc)

                acc = lax.fori_loop(0, chunk, pos_body, acc)
            return acc

        acc = lax.fori_loop(0, ch_per_p // 2, c2_body, acc)
        for g in range(ng):
            v1, v2, v3, v4, v5 = acc[TOPK * g:TOPK * (g + 1)]
            simbuf[pl.ds(g * LANES, LANES)] = (
                (v1 + v2 + v3 + v4 + v5) * jnp.float32(1.0 / TOPK))
        pltpu.sync_copy(simbuf, out.at[pstart + pi])
        return carry

    lax.fori_loop(0, cnt, p_body, 0)


def _build_sc(np_, hw, lanes, chunk):
    mesh = plsc.VectorSubcoreMesh(
        core_axis_name="c", subcore_axis_name="s",
        num_cores=NUM_CORES, num_subcores=NUM_SUBCORES)
    return pl.kernel(
        functools.partial(_sc_body, np_=np_, hw=hw, chunk=chunk, lanes=lanes),
        out_type=jax.ShapeDtypeStruct((np_, lanes), jnp.float32),
        mesh=mesh,
        scratch_types=[
            pltpu.VMEM((2, chunk, lanes), jnp.float32),
            pltpu.VMEM((lanes,), jnp.float32),
            pltpu.SemaphoreType.DMA,
            pltpu.SemaphoreType.DMA,
        ],
        compiler_params=pltpu.CompilerParams(
            needs_layout_passes=False, use_tc_tiling_on_sc=True),
    )


def _tc_linear(sim_ref, w_ref, b_ref, o_ref):
    s = sim_ref[...].astype(jnp.bfloat16).astype(jnp.float32)
    w = w_ref[...].astype(jnp.bfloat16).astype(jnp.float32)
    o_ref[...] = jnp.sum(s * w, axis=0, keepdims=True) + b_ref[...]


def kernel(prototype_activations, upsampled_activation, W, b):
    B, P = prototype_activations.shape[0], prototype_activations.shape[1]
    hw = prototype_activations.shape[2] * prototype_activations.shape[3]

    acts = jnp.transpose(prototype_activations, (1, 2, 3, 0)).reshape(P * hw, B)

    sc = _build_sc(P, hw, B, chunk=128)
    sim = sc(acts)

    logits_row = pl.pallas_call(
        _tc_linear,
        out_shape=jax.ShapeDtypeStruct((1, B), jnp.float32),
    )(sim, W.reshape(P, 1), b.reshape(1, 1))
    return logits_row.reshape(B, 1)


# speedup vs baseline: 3.4383x; 2.5206x over previous
import functools

import jax
import jax.numpy as jnp
from jax import lax
from jax.experimental import pallas as pl
from jax.experimental.pallas import tpu as pltpu
from jax.experimental.pallas import tpu_sc as plsc

NUM_CORES = 2
NUM_SUBCORES = 16
NUM_WORKERS = NUM_CORES * NUM_SUBCORES
LANES = 16

TOPK = 5
NEG = float("-inf")


def _sc_body(acts, out, buf, simbuf, sem0, sem1, *, np_, hw, chunk, lanes):
    ng = lanes // LANES
    ch_per_p = hw // chunk
    wid = lax.axis_index("s") * NUM_CORES + lax.axis_index("c")
    base, rem = np_ // NUM_WORKERS, np_ % NUM_WORKERS
    cnt = jnp.where(wid < rem, base + 1, base)
    pstart = wid * base + jnp.minimum(wid, rem)
    sems = (sem0, sem1)
    neg = jnp.full((LANES,), NEG, jnp.float32)

    def chunk_copy(ci, bslot):
        return pltpu.make_async_copy(
            acts.at[pl.ds(ci * chunk, chunk)], buf.at[bslot], sems[bslot])

    first = pstart * ch_per_p
    last = (pstart + cnt) * ch_per_p
    chunk_copy(first, 0).start()

    def p_body(pi, carry):
        acc = tuple(neg for _ in range(TOPK * ng))

        def c2_body(c2, acc):
            for bslot in range(2):
                ci = (pstart + pi) * ch_per_p + c2 * 2 + bslot
                chunk_copy(ci, bslot).wait()

                @pl.when(ci + 1 < last)
                def _():
                    chunk_copy(ci + 1, 1 - bslot).start()

                def make_pos_body(groups, unroll):
                    def pos_body(i, sub):
                        sub = list(sub)
                        for u in range(unroll):
                            for k, g in enumerate(groups):
                                x = buf[bslot, i * unroll + u,
                                        pl.ds(g * LANES, LANES)]
                                v1, v2, v3, v4, v5 = sub[TOPK * k:TOPK * (k + 1)]
                                t1 = jnp.maximum(v1, x); x = jnp.minimum(v1, x); v1 = t1
                                t1 = jnp.maximum(v2, x); x = jnp.minimum(v2, x); v2 = t1
                                t1 = jnp.maximum(v3, x); x = jnp.minimum(v3, x); v3 = t1
                                t1 = jnp.maximum(v4, x); x = jnp.minimum(v4, x); v4 = t1
                                v5 = jnp.maximum(v5, x)
                                sub[TOPK * k:TOPK * (k + 1)] = [v1, v2, v3, v4, v5]
                        return tuple(sub)
                    return pos_body

                half = ng // 2
                acc_a = lax.fori_loop(
                    0, chunk // 2, make_pos_body(range(half), 2),
                    tuple(acc[:TOPK * half]))
                acc_b = lax.fori_loop(
                    0, chunk // 2, make_pos_body(range(half, ng), 2),
                    tuple(acc[TOPK * half:]))
                acc = acc_a + acc_b
            return acc

        acc = lax.fori_loop(0, ch_per_p // 2, c2_body, acc)
        for g in range(ng):
            v1, v2, v3, v4, v5 = acc[TOPK * g:TOPK * (g + 1)]
            simbuf[pl.ds(g * LANES, LANES)] = (
                (v1 + v2 + v3 + v4 + v5) * jnp.float32(1.0 / TOPK))
        pltpu.sync_copy(simbuf, out.at[pstart + pi])
        return carry

    lax.fori_loop(0, cnt, p_body, 0)


def _build_sc(np_, hw, lanes, chunk):
    mesh = plsc.VectorSubcoreMesh(
        core_axis_name="c", subcore_axis_name="s",
        num_cores=NUM_CORES, num_subcores=NUM_SUBCORES)
    return pl.kernel(
        functools.partial(_sc_body, np_=np_, hw=hw, chunk=chunk, lanes=lanes),
        out_type=jax.ShapeDtypeStruct((np_, lanes), jnp.float32),
        mesh=mesh,
        scratch_types=[
            pltpu.VMEM((2, chunk, lanes), jnp.float32),
            pltpu.VMEM((lanes,), jnp.float32),
            pltpu.SemaphoreType.DMA,
            pltpu.SemaphoreType.DMA,
        ],
        compiler_params=pltpu.CompilerParams(
            needs_layout_passes=False, use_tc_tiling_on_sc=True),
    )


def _tc_linear(sim_ref, w_ref, b_ref, o_ref):
    s = sim_ref[...].astype(jnp.bfloat16).astype(jnp.float32)
    w = w_ref[...].astype(jnp.bfloat16).astype(jnp.float32)
    o_ref[...] = jnp.sum(s * w, axis=0, keepdims=True) + b_ref[...]


def kernel(prototype_activations, upsampled_activation, W, b):
    B, P = prototype_activations.shape[0], prototype_activations.shape[1]
    hw = prototype_activations.shape[2] * prototype_activations.shape[3]

    acts = jnp.transpose(prototype_activations, (1, 2, 3, 0)).reshape(P * hw, B)

    sc = _build_sc(P, hw, B, chunk=128)
    sim = sc(acts)

    logits_row = pl.pallas_call(
        _tc_linear,
        out_shape=jax.ShapeDtypeStruct((1, B), jnp.float32),
    )(sim, W.reshape(P, 1), b.reshape(1, 1))
    return logits_row.reshape(B, 1)


# speedup vs baseline: 4.1169x; 1.1974x over previous
import functools

import jax
import jax.numpy as jnp
from jax import lax
from jax.experimental import pallas as pl
from jax.experimental.pallas import tpu as pltpu
from jax.experimental.pallas import tpu_sc as plsc

NUM_CORES = 2
NUM_SUBCORES = 16
NUM_WORKERS = NUM_CORES * NUM_SUBCORES
LANES = 16

TOPK = 5
NEG = float("-inf")


def _sc_body(acts, out, buf, simbuf, sem0, sem1, *, np_, hw, chunk, lanes):
    ng = lanes // LANES
    ch_per_p = hw // chunk
    wid = lax.axis_index("s") * NUM_CORES + lax.axis_index("c")
    base, rem = np_ // NUM_WORKERS, np_ % NUM_WORKERS
    cnt = jnp.where(wid < rem, base + 1, base)
    pstart = wid * base + jnp.minimum(wid, rem)
    sems = (sem0, sem1)
    neg = jnp.full((LANES,), NEG, jnp.float32)

    def chunk_copy(ci, bslot):
        return pltpu.make_async_copy(
            acts.at[pl.ds(ci * chunk, chunk)], buf.at[bslot], sems[bslot])

    first = pstart * ch_per_p
    last = (pstart + cnt) * ch_per_p
    chunk_copy(first, 0).start()

    def p_body(pi, carry):
        acc = tuple(neg for _ in range(TOPK * ng))

        def c2_body(c2, acc):
            for bslot in range(2):
                ci = (pstart + pi) * ch_per_p + c2 * 2 + bslot
                chunk_copy(ci, bslot).wait()

                @pl.when(ci + 1 < last)
                def _():
                    chunk_copy(ci + 1, 1 - bslot).start()

                def make_pos_body(groups, unroll):
                    def pos_body(i, sub):
                        sub = list(sub)
                        for u in range(unroll):
                            for k, g in enumerate(groups):
                                x = buf[bslot, i * unroll + u,
                                        pl.ds(g * LANES, LANES)]
                                v1, v2, v3, v4, v5 = sub[TOPK * k:TOPK * (k + 1)]
                                t1 = jnp.maximum(v1, x); x = jnp.minimum(v1, x); v1 = t1
                                t1 = jnp.maximum(v2, x); x = jnp.minimum(v2, x); v2 = t1
                                t1 = jnp.maximum(v3, x); x = jnp.minimum(v3, x); v3 = t1
                                t1 = jnp.maximum(v4, x); x = jnp.minimum(v4, x); v4 = t1
                                v5 = jnp.maximum(v5, x)
                                sub[TOPK * k:TOPK * (k + 1)] = [v1, v2, v3, v4, v5]
                        return tuple(sub)
                    return pos_body

                half = ng // 2
                acc_a = lax.fori_loop(
                    0, chunk // 2, make_pos_body(range(half), 2),
                    tuple(acc[:TOPK * half]))
                acc_b = lax.fori_loop(
                    0, chunk // 2, make_pos_body(range(half, ng), 2),
                    tuple(acc[TOPK * half:]))
                acc = acc_a + acc_b
            return acc

        acc = lax.fori_loop(0, ch_per_p // 2, c2_body, acc)
        for g in range(ng):
            v1, v2, v3, v4, v5 = acc[TOPK * g:TOPK * (g + 1)]
            simbuf[pl.ds(g * LANES, LANES)] = (
                (v1 + v2 + v3 + v4 + v5) * jnp.float32(1.0 / TOPK))
        pltpu.sync_copy(simbuf, out.at[pstart + pi])
        return carry

    lax.fori_loop(0, cnt, p_body, 0)


def _build_sc(np_, hw, lanes, chunk):
    mesh = plsc.VectorSubcoreMesh(
        core_axis_name="c", subcore_axis_name="s",
        num_cores=NUM_CORES, num_subcores=NUM_SUBCORES)
    return pl.kernel(
        functools.partial(_sc_body, np_=np_, hw=hw, chunk=chunk, lanes=lanes),
        out_type=jax.ShapeDtypeStruct((np_, lanes), jnp.float32),
        mesh=mesh,
        scratch_types=[
            pltpu.VMEM((2, chunk, lanes), jnp.float32),
            pltpu.VMEM((lanes,), jnp.float32),
            pltpu.SemaphoreType.DMA,
            pltpu.SemaphoreType.DMA,
        ],
        compiler_params=pltpu.CompilerParams(
            needs_layout_passes=False, use_tc_tiling_on_sc=True),
    )


def _tc_topk_body(x_ref, o_ref, *, hw, lanes, unroll):
    neg = jnp.full((8, lanes), NEG, jnp.float32)

    def slab_body(i, acc):
        v1, v2, v3, v4, v5 = acc
        for u in range(unroll):
            x = x_ref[pl.ds((i * unroll + u) * 8, 8), :]
            t1 = jnp.maximum(v1, x); x = jnp.minimum(v1, x); v1 = t1
            t1 = jnp.maximum(v2, x); x = jnp.minimum(v2, x); v2 = t1
            t1 = jnp.maximum(v3, x); x = jnp.minimum(v3, x); v3 = t1
            t1 = jnp.maximum(v4, x); x = jnp.minimum(v4, x); v4 = t1
            v5 = jnp.maximum(v5, x)
        return (v1, v2, v3, v4, v5)

    v1, v2, v3, v4, v5 = lax.fori_loop(
        0, hw // 8 // unroll, slab_body, (neg,) * TOPK)

    iota8 = lax.broadcasted_iota(jnp.int32, (8, lanes), 0)
    s = jnp.zeros((1, lanes), jnp.float32)
    for _ in range(TOPK):
        m = jnp.max(v1, axis=0, keepdims=True)
        s = s + m
        eq = v1 == m
        fs = jnp.min(jnp.where(eq, iota8, 8), axis=0, keepdims=True)
        msk = iota8 == fs
        v1 = jnp.where(msk, v2, v1)
        v2 = jnp.where(msk, v3, v2)
        v3 = jnp.where(msk, v4, v3)
        v4 = jnp.where(msk, v5, v4)
    o_ref[...] = (s * jnp.float32(1.0 / TOPK))[None]


def _build_tc_topk(p0, np_, hw, lanes, unroll=2):
    return pl.pallas_call(
        functools.partial(_tc_topk_body, hw=hw, lanes=lanes, unroll=unroll),
        grid=(np_,),
        in_specs=[pl.BlockSpec((hw, lanes), lambda i: (i + p0, 0))],
        out_specs=pl.BlockSpec((1, 1, lanes), lambda i: (i, 0, 0)),
        out_shape=jax.ShapeDtypeStruct((np_, 1, lanes), jnp.float32),
    )


def _tc_linear(sim_ref, w_ref, b_ref, o_ref):
    s = sim_ref[...].astype(jnp.bfloat16).astype(jnp.float32)
    w = w_ref[...].astype(jnp.bfloat16).astype(jnp.float32)
    o_ref[...] = jnp.sum(s * w, axis=0, keepdims=True) + b_ref[...]


def kernel(prototype_activations, upsampled_activation, W, b):
    B, P = prototype_activations.shape[0], prototype_activations.shape[1]
    hw = prototype_activations.shape[2] * prototype_activations.shape[3]

    acts = jnp.transpose(prototype_activations, (1, 2, 3, 0)).reshape(P * hw, B)

    sp = NUM_WORKERS if P >= 2 * NUM_WORKERS else P
    sc = _build_sc(sp, hw, B, chunk=128)
    sim_sc = sc(acts)
    if sp < P:
        sim_tc = _build_tc_topk(sp, P - sp, hw, B)(acts).reshape(P - sp, B)
        sim = jnp.concatenate([sim_sc, sim_tc], axis=0)
    else:
        sim = sim_sc

    logits_row = pl.pallas_call(
        _tc_linear,
        out_shape=jax.ShapeDtypeStruct((1, B), jnp.float32),
    )(sim, W.reshape(P, 1), b.reshape(1, 1))
    return logits_row.reshape(B, 1)


# speedup vs baseline: 5.7935x; 1.4073x over previous
import functools

import jax
import jax.numpy as jnp
from jax import lax
from jax.experimental import pallas as pl
from jax.experimental.pallas import tpu as pltpu
from jax.experimental.pallas import tpu_sc as plsc

NUM_CORES = 2
NUM_SUBCORES = 16
NUM_WORKERS = NUM_CORES * NUM_SUBCORES
LANES = 16

TOPK = 5
NEG = float("-inf")


def _sc_body(acts, out, buf, simbuf, sem0, sem1, *, np_, hw, chunk, lanes):
    ng = lanes // LANES
    ch_per_p = hw // chunk
    wid = lax.axis_index("s") * NUM_CORES + lax.axis_index("c")
    base, rem = np_ // NUM_WORKERS, np_ % NUM_WORKERS
    cnt = jnp.where(wid < rem, base + 1, base)
    pstart = wid * base + jnp.minimum(wid, rem)
    sems = (sem0, sem1)
    neg = jnp.full((LANES,), NEG, jnp.float32)

    def chunk_copy(ci, bslot):
        return pltpu.make_async_copy(
            acts.at[pl.ds(ci * chunk, chunk)], buf.at[bslot], sems[bslot])

    first = pstart * ch_per_p
    last = (pstart + cnt) * ch_per_p
    chunk_copy(first, 0).start()

    def p_body(pi, carry):
        acc = tuple(neg for _ in range(TOPK * ng))

        def c2_body(c2, acc):
            for bslot in range(2):
                ci = (pstart + pi) * ch_per_p + c2 * 2 + bslot
                chunk_copy(ci, bslot).wait()

                @pl.when(ci + 1 < last)
                def _():
                    chunk_copy(ci + 1, 1 - bslot).start()

                def make_pos_body(groups, unroll):
                    def pos_body(i, sub):
                        sub = list(sub)
                        for u in range(unroll):
                            for k, g in enumerate(groups):
                                x = buf[bslot, i * unroll + u,
                                        pl.ds(g * LANES, LANES)]
                                v1, v2, v3, v4, v5 = sub[TOPK * k:TOPK * (k + 1)]
                                t1 = jnp.maximum(v1, x); x = jnp.minimum(v1, x); v1 = t1
                                t1 = jnp.maximum(v2, x); x = jnp.minimum(v2, x); v2 = t1
                                t1 = jnp.maximum(v3, x); x = jnp.minimum(v3, x); v3 = t1
                                t1 = jnp.maximum(v4, x); x = jnp.minimum(v4, x); v4 = t1
                                v5 = jnp.maximum(v5, x)
                                sub[TOPK * k:TOPK * (k + 1)] = [v1, v2, v3, v4, v5]
                        return tuple(sub)
                    return pos_body

                half = ng // 2
                acc_a = lax.fori_loop(
                    0, chunk // 2, make_pos_body(range(half), 2),
                    tuple(acc[:TOPK * half]))
                acc_b = lax.fori_loop(
                    0, chunk // 2, make_pos_body(range(half, ng), 2),
                    tuple(acc[TOPK * half:]))
                acc = acc_a + acc_b
            return acc

        acc = lax.fori_loop(0, ch_per_p // 2, c2_body, acc)
        for g in range(ng):
            v1, v2, v3, v4, v5 = acc[TOPK * g:TOPK * (g + 1)]
            simbuf[pl.ds(g * LANES, LANES)] = (
                (v1 + v2 + v3 + v4 + v5) * jnp.float32(1.0 / TOPK))
        pltpu.sync_copy(simbuf, out.at[pstart + pi])
        return carry

    lax.fori_loop(0, cnt, p_body, 0)


def _build_sc(np_, hw, lanes, chunk):
    mesh = plsc.VectorSubcoreMesh(
        core_axis_name="c", subcore_axis_name="s",
        num_cores=NUM_CORES, num_subcores=NUM_SUBCORES)
    return pl.kernel(
        functools.partial(_sc_body, np_=np_, hw=hw, chunk=chunk, lanes=lanes),
        out_type=jax.ShapeDtypeStruct((np_, lanes), jnp.float32),
        mesh=mesh,
        scratch_types=[
            pltpu.VMEM((2, chunk, lanes), jnp.float32),
            pltpu.VMEM((lanes,), jnp.float32),
            pltpu.SemaphoreType.DMA,
            pltpu.SemaphoreType.DMA,
        ],
        compiler_params=pltpu.CompilerParams(
            needs_layout_passes=False, use_tc_tiling_on_sc=True),
    )


def _tc_topk_body(x_ref, o_ref, *, hw, lanes, slab):
    neg = jnp.full((slab, lanes), NEG, jnp.float32)

    def slab_body(i, acc):
        v1, v2, v3, v4, v5 = acc
        x = x_ref[pl.ds(i * slab, slab), :]
        t1 = jnp.maximum(v1, x); x = jnp.minimum(v1, x); v1 = t1
        t1 = jnp.maximum(v2, x); x = jnp.minimum(v2, x); v2 = t1
        t1 = jnp.maximum(v3, x); x = jnp.minimum(v3, x); v3 = t1
        t1 = jnp.maximum(v4, x); x = jnp.minimum(v4, x); v4 = t1
        v5 = jnp.maximum(v5, x)
        return (v1, v2, v3, v4, v5)

    v1, v2, v3, v4, v5 = lax.fori_loop(0, hw // slab, slab_body, (neg,) * TOPK)

    iota = lax.broadcasted_iota(jnp.int32, (slab, lanes), 0)
    s = jnp.zeros((1, lanes), jnp.float32)
    for _ in range(TOPK):
        m = jnp.max(v1, axis=0, keepdims=True)
        s = s + m
        eq = v1 == m
        fs = jnp.min(jnp.where(eq, iota, slab), axis=0, keepdims=True)
        msk = iota == fs
        v1 = jnp.where(msk, v2, v1)
        v2 = jnp.where(msk, v3, v2)
        v3 = jnp.where(msk, v4, v3)
        v4 = jnp.where(msk, v5, v4)
    o_ref[...] = (s * jnp.float32(1.0 / TOPK))[None]


def _build_tc_topk(p0, np_, hw, lanes, slab=32):
    return pl.pallas_call(
        functools.partial(_tc_topk_body, hw=hw, lanes=lanes, slab=slab),
        grid=(np_,),
        in_specs=[pl.BlockSpec((hw, lanes), lambda i: (i + p0, 0))],
        out_specs=pl.BlockSpec((1, 1, lanes), lambda i: (i, 0, 0)),
        out_shape=jax.ShapeDtypeStruct((np_, 1, lanes), jnp.float32),
    )


def _tc_linear(sim_ref, w_ref, b_ref, o_ref):
    s = sim_ref[...].astype(jnp.bfloat16).astype(jnp.float32)
    w = w_ref[...].astype(jnp.bfloat16).astype(jnp.float32)
    o_ref[...] = jnp.sum(s * w, axis=0, keepdims=True) + b_ref[...]


def kernel(prototype_activations, upsampled_activation, W, b):
    B, P = prototype_activations.shape[0], prototype_activations.shape[1]
    hw = prototype_activations.shape[2] * prototype_activations.shape[3]

    acts = jnp.transpose(prototype_activations, (1, 2, 3, 0)).reshape(P * hw, B)

    sp = NUM_WORKERS if P >= 2 * NUM_WORKERS else P
    sc = _build_sc(sp, hw, B, chunk=128)
    sim_sc = sc(acts)
    if sp < P:
        sim_tc = _build_tc_topk(sp, P - sp, hw, B)(acts).reshape(P - sp, B)
        sim = jnp.concatenate([sim_sc, sim_tc], axis=0)
    else:
        sim = sim_sc

    logits_row = pl.pallas_call(
        _tc_linear,
        out_shape=jax.ShapeDtypeStruct((1, B), jnp.float32),
    )(sim, W.reshape(P, 1), b.reshape(1, 1))
    return logits_row.reshape(B, 1)


# speedup vs baseline: 6.1483x; 1.0612x over previous
import functools

import jax
import jax.numpy as jnp
from jax import lax
from jax.experimental import pallas as pl
from jax.experimental.pallas import tpu as pltpu
from jax.experimental.pallas import tpu_sc as plsc

NUM_CORES = 2
NUM_SUBCORES = 16
NUM_WORKERS = NUM_CORES * NUM_SUBCORES
LANES = 16

TOPK = 5
NEG = float("-inf")


def _sc_body(acts, out, buf, simbuf, sem0, sem1, *, np_, hw, chunk, lanes):
    ng = lanes // LANES
    ch_per_p = hw // chunk
    wid = lax.axis_index("s") * NUM_CORES + lax.axis_index("c")
    base, rem = np_ // NUM_WORKERS, np_ % NUM_WORKERS
    cnt = jnp.where(wid < rem, base + 1, base)
    pstart = wid * base + jnp.minimum(wid, rem)
    sems = (sem0, sem1)
    neg = jnp.full((LANES,), NEG, jnp.float32)

    def chunk_copy(ci, bslot):
        return pltpu.make_async_copy(
            acts.at[pl.ds(ci * chunk, chunk)], buf.at[bslot], sems[bslot])

    first = pstart * ch_per_p
    last = (pstart + cnt) * ch_per_p
    chunk_copy(first, 0).start()

    def p_body(pi, carry):
        acc = tuple(neg for _ in range(TOPK * ng))

        def c2_body(c2, acc):
            for bslot in range(2):
                ci = (pstart + pi) * ch_per_p + c2 * 2 + bslot
                chunk_copy(ci, bslot).wait()

                @pl.when(ci + 1 < last)
                def _():
                    chunk_copy(ci + 1, 1 - bslot).start()

                def make_pos_body(groups, unroll):
                    def pos_body(i, sub):
                        sub = list(sub)
                        for u in range(unroll):
                            for k, g in enumerate(groups):
                                x = buf[bslot, i * unroll + u,
                                        pl.ds(g * LANES, LANES)]
                                v1, v2, v3, v4, v5 = sub[TOPK * k:TOPK * (k + 1)]
                                t1 = jnp.maximum(v1, x); x = jnp.minimum(v1, x); v1 = t1
                                t1 = jnp.maximum(v2, x); x = jnp.minimum(v2, x); v2 = t1
                                t1 = jnp.maximum(v3, x); x = jnp.minimum(v3, x); v3 = t1
                                t1 = jnp.maximum(v4, x); x = jnp.minimum(v4, x); v4 = t1
                                v5 = jnp.maximum(v5, x)
                                sub[TOPK * k:TOPK * (k + 1)] = [v1, v2, v3, v4, v5]
                        return tuple(sub)
                    return pos_body

                half = ng // 2
                acc_a = lax.fori_loop(
                    0, chunk // 2, make_pos_body(range(half), 2),
                    tuple(acc[:TOPK * half]))
                acc_b = lax.fori_loop(
                    0, chunk // 2, make_pos_body(range(half, ng), 2),
                    tuple(acc[TOPK * half:]))
                acc = acc_a + acc_b
            return acc

        acc = lax.fori_loop(0, ch_per_p // 2, c2_body, acc)
        for g in range(ng):
            v1, v2, v3, v4, v5 = acc[TOPK * g:TOPK * (g + 1)]
            simbuf[pl.ds(g * LANES, LANES)] = (
                (v1 + v2 + v3 + v4 + v5) * jnp.float32(1.0 / TOPK))
        pltpu.sync_copy(simbuf, out.at[pstart + pi])
        return carry

    lax.fori_loop(0, cnt, p_body, 0)


def _build_sc(np_, hw, lanes, chunk):
    mesh = plsc.VectorSubcoreMesh(
        core_axis_name="c", subcore_axis_name="s",
        num_cores=NUM_CORES, num_subcores=NUM_SUBCORES)
    return pl.kernel(
        functools.partial(_sc_body, np_=np_, hw=hw, chunk=chunk, lanes=lanes),
        out_type=jax.ShapeDtypeStruct((np_, lanes), jnp.float32),
        mesh=mesh,
        scratch_types=[
            pltpu.VMEM((2, chunk, lanes), jnp.float32),
            pltpu.VMEM((lanes,), jnp.float32),
            pltpu.SemaphoreType.DMA,
            pltpu.SemaphoreType.DMA,
        ],
        compiler_params=pltpu.CompilerParams(
            needs_layout_passes=False, use_tc_tiling_on_sc=True),
    )


def _tc_topk_body(*refs, hw, lanes, slab, nstream):
    x_refs, o_ref = refs[:-1], refs[-1]
    neg = jnp.full((slab, lanes), NEG, jnp.float32)
    rows = hw // nstream

    def slab_body(i, acc):
        v1, v2, v3, v4, v5 = acc
        for r in x_refs:
            x = r[pl.ds(i * slab, slab), :]
            t1 = jnp.maximum(v1, x); x = jnp.minimum(v1, x); v1 = t1
            t1 = jnp.maximum(v2, x); x = jnp.minimum(v2, x); v2 = t1
            t1 = jnp.maximum(v3, x); x = jnp.minimum(v3, x); v3 = t1
            t1 = jnp.maximum(v4, x); x = jnp.minimum(v4, x); v4 = t1
            v5 = jnp.maximum(v5, x)
        return (v1, v2, v3, v4, v5)

    v1, v2, v3, v4, v5 = lax.fori_loop(0, rows // slab, slab_body, (neg,) * TOPK)

    iota = lax.broadcasted_iota(jnp.int32, (slab, lanes), 0)
    s = jnp.zeros((1, lanes), jnp.float32)
    for _ in range(TOPK):
        m = jnp.max(v1, axis=0, keepdims=True)
        s = s + m
        eq = v1 == m
        fs = jnp.min(jnp.where(eq, iota, slab), axis=0, keepdims=True)
        msk = iota == fs
        v1 = jnp.where(msk, v2, v1)
        v2 = jnp.where(msk, v3, v2)
        v3 = jnp.where(msk, v4, v3)
        v4 = jnp.where(msk, v5, v4)
    o_ref[...] = (s * jnp.float32(1.0 / TOPK))[None]


def _build_tc_topk(p0, np_, hw, lanes, slab=32, nstream=2):
    rows = hw // nstream
    specs = [
        pl.BlockSpec((rows, lanes),
                     functools.partial(lambda s, i: (nstream * (i + p0) + s, 0), s))
        for s in range(nstream)
    ]
    return pl.pallas_call(
        functools.partial(_tc_topk_body, hw=hw, lanes=lanes, slab=slab,
                          nstream=nstream),
        grid=(np_,),
        in_specs=specs,
        out_specs=pl.BlockSpec((1, 1, lanes), lambda i: (i, 0, 0)),
        out_shape=jax.ShapeDtypeStruct((np_, 1, lanes), jnp.float32),
    )


def _tc_linear(sim_ref, w_ref, b_ref, o_ref):
    s = sim_ref[...].astype(jnp.bfloat16).astype(jnp.float32)
    w = w_ref[...].astype(jnp.bfloat16).astype(jnp.float32)
    o_ref[...] = jnp.sum(s * w, axis=0, keepdims=True) + b_ref[...]


def kernel(prototype_activations, upsampled_activation, W, b):
    B, P = prototype_activations.shape[0], prototype_activations.shape[1]
    hw = prototype_activations.shape[2] * prototype_activations.shape[3]

    acts = jnp.transpose(prototype_activations, (1, 2, 3, 0)).reshape(P * hw, B)

    sp = NUM_WORKERS if P >= 2 * NUM_WORKERS else P
    sc = _build_sc(sp, hw, B, chunk=128)
    sim_sc = sc(acts)
    if sp < P:
        sim_tc = _build_tc_topk(sp, P - sp, hw, B)(acts, acts).reshape(P - sp, B)
        sim = jnp.concatenate([sim_sc, sim_tc], axis=0)
    else:
        sim = sim_sc

    logits_row = pl.pallas_call(
        _tc_linear,
        out_shape=jax.ShapeDtypeStruct((1, B), jnp.float32),
    )(sim, W.reshape(P, 1), b.reshape(1, 1))
    return logits_row.reshape(B, 1)


# speedup vs baseline: 6.4988x; 1.0570x over previous
import functools

import jax
import jax.numpy as jnp
from jax import lax
from jax.experimental import pallas as pl
from jax.experimental.pallas import tpu as pltpu
from jax.experimental.pallas import tpu_sc as plsc

NUM_CORES = 2
NUM_SUBCORES = 16
NUM_WORKERS = NUM_CORES * NUM_SUBCORES
LANES = 16

TOPK = 5
NEG = float("-inf")


def _sc_body(acts, out, buf, simbuf, sem0, sem1, *, np_, hw, chunk, lanes):
    ng = lanes // LANES
    ch_per_p = hw // chunk
    wid = lax.axis_index("s") * NUM_CORES + lax.axis_index("c")
    base, rem = np_ // NUM_WORKERS, np_ % NUM_WORKERS
    cnt = jnp.where(wid < rem, base + 1, base)
    pstart = wid * base + jnp.minimum(wid, rem)
    sems = (sem0, sem1)
    neg = jnp.full((LANES,), NEG, jnp.float32)

    def chunk_copy(ci, bslot):
        return pltpu.make_async_copy(
            acts.at[pl.ds(ci * chunk, chunk)], buf.at[bslot], sems[bslot])

    first = pstart * ch_per_p
    last = (pstart + cnt) * ch_per_p
    chunk_copy(first, 0).start()

    def p_body(pi, carry):
        acc = tuple(neg for _ in range(TOPK * ng))

        def c2_body(c2, acc):
            for bslot in range(2):
                ci = (pstart + pi) * ch_per_p + c2 * 2 + bslot
                chunk_copy(ci, bslot).wait()

                @pl.when(ci + 1 < last)
                def _():
                    chunk_copy(ci + 1, 1 - bslot).start()

                def make_pos_body(groups, unroll):
                    def pos_body(i, sub):
                        sub = list(sub)
                        for u in range(unroll):
                            for k, g in enumerate(groups):
                                x = buf[bslot, i * unroll + u,
                                        pl.ds(g * LANES, LANES)]
                                v1, v2, v3, v4, v5 = sub[TOPK * k:TOPK * (k + 1)]
                                t1 = jnp.maximum(v1, x); x = jnp.minimum(v1, x); v1 = t1
                                t1 = jnp.maximum(v2, x); x = jnp.minimum(v2, x); v2 = t1
                                t1 = jnp.maximum(v3, x); x = jnp.minimum(v3, x); v3 = t1
                                t1 = jnp.maximum(v4, x); x = jnp.minimum(v4, x); v4 = t1
                                v5 = jnp.maximum(v5, x)
                                sub[TOPK * k:TOPK * (k + 1)] = [v1, v2, v3, v4, v5]
                        return tuple(sub)
                    return pos_body

                half = ng // 2
                acc_a = lax.fori_loop(
                    0, chunk // 2, make_pos_body(range(half), 2),
                    tuple(acc[:TOPK * half]))
                acc_b = lax.fori_loop(
                    0, chunk // 2, make_pos_body(range(half, ng), 2),
                    tuple(acc[TOPK * half:]))
                acc = acc_a + acc_b
            return acc

        acc = lax.fori_loop(0, ch_per_p // 2, c2_body, acc)
        for g in range(ng):
            v1, v2, v3, v4, v5 = acc[TOPK * g:TOPK * (g + 1)]
            simbuf[pl.ds(g * LANES, LANES)] = (
                (v1 + v2 + v3 + v4 + v5) * jnp.float32(1.0 / TOPK))
        pltpu.sync_copy(simbuf, out.at[pstart + pi])
        return carry

    lax.fori_loop(0, cnt, p_body, 0)


def _build_sc(np_, hw, lanes, chunk):
    mesh = plsc.VectorSubcoreMesh(
        core_axis_name="c", subcore_axis_name="s",
        num_cores=NUM_CORES, num_subcores=NUM_SUBCORES)
    return pl.kernel(
        functools.partial(_sc_body, np_=np_, hw=hw, chunk=chunk, lanes=lanes),
        out_type=jax.ShapeDtypeStruct((np_, lanes), jnp.float32),
        mesh=mesh,
        scratch_types=[
            pltpu.VMEM((2, chunk, lanes), jnp.float32),
            pltpu.VMEM((lanes,), jnp.float32),
            pltpu.SemaphoreType.DMA,
            pltpu.SemaphoreType.DMA,
        ],
        compiler_params=pltpu.CompilerParams(
            needs_layout_passes=False, use_tc_tiling_on_sc=True),
    )


def _tc_topk_body(*refs, hw, lanes, slab, nstream):
    x_refs, o_ref = refs[:-1], refs[-1]
    neg = jnp.full((slab, lanes), NEG, jnp.float32)
    rows = hw // nstream

    def slab_body(i, acc):
        v1, v2, v3, v4, v5 = acc
        for r in x_refs:
            x = r[pl.ds(i * slab, slab), :]
            t1 = jnp.maximum(v1, x); x = jnp.minimum(v1, x); v1 = t1
            t1 = jnp.maximum(v2, x); x = jnp.minimum(v2, x); v2 = t1
            t1 = jnp.maximum(v3, x); x = jnp.minimum(v3, x); v3 = t1
            t1 = jnp.maximum(v4, x); x = jnp.minimum(v4, x); v4 = t1
            v5 = jnp.maximum(v5, x)
        return (v1, v2, v3, v4, v5)

    v1, v2, v3, v4, v5 = lax.fori_loop(0, rows // slab, slab_body, (neg,) * TOPK)

    iota = lax.broadcasted_iota(jnp.int32, (slab, lanes), 0)
    s = jnp.zeros((1, lanes), jnp.float32)
    for _ in range(TOPK):
        m = jnp.max(v1, axis=0, keepdims=True)
        s = s + m
        eq = v1 == m
        fs = jnp.min(jnp.where(eq, iota, slab), axis=0, keepdims=True)
        msk = iota == fs
        v1 = jnp.where(msk, v2, v1)
        v2 = jnp.where(msk, v3, v2)
        v3 = jnp.where(msk, v4, v3)
        v4 = jnp.where(msk, v5, v4)
    o_ref[...] = (s * jnp.float32(1.0 / TOPK))[None]


def _build_tc_topk(p0, np_, hw, lanes, slab=32, nstream=2):
    rows = hw // nstream
    specs = [
        pl.BlockSpec((rows, lanes),
                     functools.partial(lambda s, i: (nstream * (i + p0) + s, 0), s))
        for s in range(nstream)
    ]
    return pl.pallas_call(
        functools.partial(_tc_topk_body, hw=hw, lanes=lanes, slab=slab,
                          nstream=nstream),
        grid=(np_,),
        in_specs=specs,
        out_specs=pl.BlockSpec((1, 1, lanes), lambda i: (i, 0, 0)),
        out_shape=jax.ShapeDtypeStruct((np_, 1, lanes), jnp.float32),
    )


def _tc_linear(sim_ref, w_ref, b_ref, o_ref):
    s = sim_ref[...].astype(jnp.bfloat16).astype(jnp.float32)
    w = w_ref[...].astype(jnp.bfloat16).astype(jnp.float32)
    o_ref[...] = jnp.sum(s * w, axis=0, keepdims=True) + b_ref[...]


def kernel(prototype_activations, upsampled_activation, W, b):
    B, P = prototype_activations.shape[0], prototype_activations.shape[1]
    hw = prototype_activations.shape[2] * prototype_activations.shape[3]

    acts = jnp.transpose(prototype_activations, (1, 2, 3, 0)).reshape(P * hw, B)

    sp = NUM_WORKERS if P >= 2 * NUM_WORKERS else P
    sc = _build_sc(sp, hw, B, chunk=128)
    sim_sc = sc(acts)
    if sp < P:
        sim_tc = _build_tc_topk(sp, P - sp, hw, B, nstream=4)(
            acts, acts, acts, acts).reshape(P - sp, B)
        sim = jnp.concatenate([sim_sc, sim_tc], axis=0)
    else:
        sim = sim_sc

    logits_row = pl.pallas_call(
        _tc_linear,
        out_shape=jax.ShapeDtypeStruct((1, B), jnp.float32),
    )(sim, W.reshape(P, 1), b.reshape(1, 1))
    return logits_row.reshape(B, 1)


# speedup vs baseline: 6.6533x; 1.0238x over previous
import functools

import jax
import jax.numpy as jnp
from jax import lax
from jax.experimental import pallas as pl
from jax.experimental.pallas import tpu as pltpu
from jax.experimental.pallas import tpu_sc as plsc

NUM_CORES = 2
NUM_SUBCORES = 16
NUM_WORKERS = NUM_CORES * NUM_SUBCORES
LANES = 16

TOPK = 5
NEG = float("-inf")


def _sc_body(acts, out, buf, simbuf, sem0, sem1, *, np_, hw, chunk, lanes):
    ng = lanes // LANES
    ch_per_p = hw // chunk
    wid = lax.axis_index("s") * NUM_CORES + lax.axis_index("c")
    base, rem = np_ // NUM_WORKERS, np_ % NUM_WORKERS
    cnt = jnp.where(wid < rem, base + 1, base)
    pstart = wid * base + jnp.minimum(wid, rem)
    sems = (sem0, sem1)
    neg = jnp.full((LANES,), NEG, jnp.float32)

    def chunk_copy(ci, bslot):
        return pltpu.make_async_copy(
            acts.at[pl.ds(ci * chunk, chunk)], buf.at[bslot], sems[bslot])

    first = pstart * ch_per_p
    last = (pstart + cnt) * ch_per_p
    chunk_copy(first, 0).start()

    def p_body(pi, carry):
        acc = tuple(neg for _ in range(TOPK * ng))

        def c2_body(c2, acc):
            for bslot in range(2):
                ci = (pstart + pi) * ch_per_p + c2 * 2 + bslot
                chunk_copy(ci, bslot).wait()

                @pl.when(ci + 1 < last)
                def _():
                    chunk_copy(ci + 1, 1 - bslot).start()

                def make_pos_body(groups, unroll):
                    def pos_body(i, sub):
                        sub = list(sub)
                        for u in range(unroll):
                            for k, g in enumerate(groups):
                                x = buf[bslot, i * unroll + u,
                                        pl.ds(g * LANES, LANES)]
                                v1, v2, v3, v4, v5 = sub[TOPK * k:TOPK * (k + 1)]
                                t1 = jnp.maximum(v1, x); x = jnp.minimum(v1, x); v1 = t1
                                t1 = jnp.maximum(v2, x); x = jnp.minimum(v2, x); v2 = t1
                                t1 = jnp.maximum(v3, x); x = jnp.minimum(v3, x); v3 = t1
                                t1 = jnp.maximum(v4, x); x = jnp.minimum(v4, x); v4 = t1
                                v5 = jnp.maximum(v5, x)
                                sub[TOPK * k:TOPK * (k + 1)] = [v1, v2, v3, v4, v5]
                        return tuple(sub)
                    return pos_body

                half = ng // 2
                acc_a = lax.fori_loop(
                    0, chunk // 2, make_pos_body(range(half), 2),
                    tuple(acc[:TOPK * half]))
                acc_b = lax.fori_loop(
                    0, chunk // 2, make_pos_body(range(half, ng), 2),
                    tuple(acc[TOPK * half:]))
                acc = acc_a + acc_b
            return acc

        acc = lax.fori_loop(0, ch_per_p // 2, c2_body, acc)
        for g in range(ng):
            v1, v2, v3, v4, v5 = acc[TOPK * g:TOPK * (g + 1)]
            simbuf[pl.ds(g * LANES, LANES)] = (
                (v1 + v2 + v3 + v4 + v5) * jnp.float32(1.0 / TOPK))
        pltpu.sync_copy(simbuf, out.at[pstart + pi])
        return carry

    lax.fori_loop(0, cnt, p_body, 0)


def _build_sc(np_, hw, lanes, chunk):
    mesh = plsc.VectorSubcoreMesh(
        core_axis_name="c", subcore_axis_name="s",
        num_cores=NUM_CORES, num_subcores=NUM_SUBCORES)
    return pl.kernel(
        functools.partial(_sc_body, np_=np_, hw=hw, chunk=chunk, lanes=lanes),
        out_type=jax.ShapeDtypeStruct((np_, lanes), jnp.float32),
        mesh=mesh,
        scratch_types=[
            pltpu.VMEM((2, chunk, lanes), jnp.float32),
            pltpu.VMEM((lanes,), jnp.float32),
            pltpu.SemaphoreType.DMA,
            pltpu.SemaphoreType.DMA,
        ],
        compiler_params=pltpu.CompilerParams(
            needs_layout_passes=False, use_tc_tiling_on_sc=True),
    )


def _tc_topk_body(*refs, hw, lanes, slab, nstream):
    x_refs, o_ref = refs[:-1], refs[-1]
    neg = jnp.full((slab, lanes), NEG, jnp.float32)
    rows = hw // nstream

    def slab_body(i, acc):
        v1, v2, v3, v4, v5 = acc
        for r in x_refs:
            x = r[pl.ds(i * slab, slab), :]
            t1 = jnp.maximum(v1, x); x = jnp.minimum(v1, x); v1 = t1
            t1 = jnp.maximum(v2, x); x = jnp.minimum(v2, x); v2 = t1
            t1 = jnp.maximum(v3, x); x = jnp.minimum(v3, x); v3 = t1
            t1 = jnp.maximum(v4, x); x = jnp.minimum(v4, x); v4 = t1
            v5 = jnp.maximum(v5, x)
        return (v1, v2, v3, v4, v5)

    v1, v2, v3, v4, v5 = lax.fori_loop(0, rows // slab, slab_body, (neg,) * TOPK)

    iota = lax.broadcasted_iota(jnp.int32, (slab, lanes), 0)
    s = jnp.zeros((1, lanes), jnp.float32)
    for _ in range(TOPK):
        m = jnp.max(v1, axis=0, keepdims=True)
        s = s + m
        eq = v1 == m
        fs = jnp.min(jnp.where(eq, iota, slab), axis=0, keepdims=True)
        msk = iota == fs
        v1 = jnp.where(msk, v2, v1)
        v2 = jnp.where(msk, v3, v2)
        v3 = jnp.where(msk, v4, v3)
        v4 = jnp.where(msk, v5, v4)
    o_ref[...] = (s * jnp.float32(1.0 / TOPK))[None]


def _build_tc_topk(p0, np_, hw, lanes, slab=32, nstream=2):
    rows = hw // nstream
    specs = [
        pl.BlockSpec((rows, lanes),
                     functools.partial(lambda s, i: (nstream * (i + p0) + s, 0), s))
        for s in range(nstream)
    ]
    return pl.pallas_call(
        functools.partial(_tc_topk_body, hw=hw, lanes=lanes, slab=slab,
                          nstream=nstream),
        grid=(np_,),
        in_specs=specs,
        out_specs=pl.BlockSpec((1, 1, lanes), lambda i: (i, 0, 0)),
        out_shape=jax.ShapeDtypeStruct((np_, 1, lanes), jnp.float32),
    )


def _tc_linear(sim_ref, w_ref, b_ref, o_ref):
    s = sim_ref[...].astype(jnp.bfloat16).astype(jnp.float32)
    w = w_ref[...].astype(jnp.bfloat16).astype(jnp.float32)
    o_ref[...] = jnp.sum(s * w, axis=0, keepdims=True) + b_ref[...]


def kernel(prototype_activations, upsampled_activation, W, b):
    B, P = prototype_activations.shape[0], prototype_activations.shape[1]
    hw = prototype_activations.shape[2] * prototype_activations.shape[3]

    acts = jnp.transpose(prototype_activations, (1, 2, 3, 0)).reshape(P * hw, B)

    sp = NUM_WORKERS if P >= 2 * NUM_WORKERS else P
    sc = _build_sc(sp, hw, B, chunk=128)
    sim_sc = sc(acts)
    if sp < P:
        ns = 8
        sim_tc = _build_tc_topk(sp, P - sp, hw, B, nstream=ns)(
            *([acts] * ns)).reshape(P - sp, B)
        sim = jnp.concatenate([sim_sc, sim_tc], axis=0)
    else:
        sim = sim_sc

    logits_row = pl.pallas_call(
        _tc_linear,
        out_shape=jax.ShapeDtypeStruct((1, B), jnp.float32),
    )(sim, W.reshape(P, 1), b.reshape(1, 1))
    return logits_row.reshape(B, 1)
